# Initial kernel scaffold; baseline (speedup 1.0000x reference)
import functools

import jax
import jax.numpy as jnp
from jax import lax
from jax.experimental import pallas as pl
from jax.experimental.pallas import tpu as pltpu
from jax.experimental.pallas import tpu_sc as plsc

N_NEURONS = 16384
N_IN = 16384
NNZ = 268435
N_BASIS = 5
N_SYN = 10
T = 256

RB = 256
NBLK = N_NEURONS // RB
NWORK = 32
PASSES = NBLK // NWORK
C = 32
NBC = T // C
S = 1024
SG = 8
NNZ_P = ((NNZ + 8 + S - 1) // S + 1) * S


def _lane_bcast(v, i):
    return lax.gather(
        v,
        jnp.full((16, 1), i, dtype=jnp.int32),
        lax.GatherDimensionNumbers(
            offset_dims=(), collapsed_slice_dims=(0,), start_index_map=(0,)),
        slice_sizes=(1,),
        mode=lax.GatherScatterMode.PROMISE_IN_BOUNDS)


def _sc_body(xcc, colsp, laddrp, wp, offs, y3,
             offs_s, cidx, la_v, w_v, xbuf, acc, sem):
    wid = lax.axis_index("c") * 16 + lax.axis_index("s")
    pltpu.sync_copy(offs, offs_s)
    ciota = lax.iota(jnp.int32, 16)

    def pass_body(p, _):
        blk = wid * PASSES + p
        p0 = offs_s[blk]
        p1 = offs_s[blk + 1]
        p0a = p0 & ~7
        nch = (p1 - p0a + S - 1) // S

        def nb_body(nb, _):
            def zr(rr, _):
                def zc(cz, _):
                    acc[rr, pl.ds(cz * 16, 16)] = jnp.zeros((16,), jnp.float32)
                    return 0
                return lax.fori_loop(0, acc.shape[1] // 16, zc, 0)
            lax.fori_loop(0, acc.shape[0], zr, 0)

            nbase = nb * N_IN

            def ch_body(ch, _):
                g0 = p0a + ch * S
                pltpu.sync_copy(colsp.at[pl.ds(g0, S)], cidx)
                pltpu.sync_copy(laddrp.at[pl.ds(g0, S)], la_v)
                pltpu.sync_copy(wp.at[pl.ds(g0, S)], w_v)

                def fix(j, _):
                    sl = pl.ds(j * 16, 16)
                    gidx = g0 + j * 16 + ciota
                    ok = (gidx >= p0) & (gidx < p1)
                    w_v[sl] = jnp.where(ok, w_v[sl], 0.0)
                    k = j // 8
                    j2 = j % 8
                    sl2 = pl.ds(j2 * 16, 16)
                    cidx[k, sl2] = cidx[k, sl2] + nbase
                    return 0
                lax.fori_loop(0, S // 16, fix, 0)

                cps = [pltpu.async_copy(xcc.at[cidx.at[k]], xbuf.at[k], sem)
                       for k in range(SG)]
                for cp in cps:
                    cp.wait()

                def grp(k, _):
                    def grp2(jj, _):
                        base = k * 128 + jj * 16
                        a16 = la_v[pl.ds(base, 16)]
                        w16 = w_v[pl.ds(base, 16)]
                        for i in range(16):
                            ab = _lane_bcast(a16, i)
                            wb = _lane_bcast(w16, i)
                            for cc in range(2):
                                xrow = xbuf[k, jj * 16 + i, pl.ds(cc * 16, 16)]
                                plsc.addupdate_scatter(
                                    acc, [ciota + cc * 16, ab], wb * xrow)
                        return 0
                    return lax.fori_loop(0, 8, grp2, 0)
                lax.fori_loop(0, SG, grp, 0)
                return 0
            lax.fori_loop(0, nch, ch_body, 0)

            for s in range(N_SYN):
                pltpu.sync_copy(
                    acc.at[:, pl.ds(s * RB, RB)],
                    y3.at[s, pl.ds(nb * C, C), pl.ds(blk * RB, RB)])
            return 0
        lax.fori_loop(0, NBC, nb_body, 0)
        return 0
    lax.fori_loop(0, PASSES, pass_body, 0)


def _tc_contract_kernel(y_ref, f_ref, o_ref):
    for r in range(N_BASIS):
        z = f_ref[0, r] * y_ref[0]
        for s in range(1, N_SYN):
            z = z + f_ref[s, r] * y_ref[s]
        o_ref[r] = z


def kernel(inp, weights, synaptic_weights, indices, syn_ids):
    b, t, n_in = inp.shape
    rows = indices[:, 0].astype(jnp.int32)
    cols = indices[:, 1].astype(jnp.int32)

    xT = inp.reshape(T, N_IN)
    xcc = xT.reshape(NBC, C, N_IN).transpose(0, 2, 1).reshape(NBC * N_IN, C)

    laddr = syn_ids.astype(jnp.int32) * RB + (rows % RB)
    pad = NNZ_P - NNZ
    colsp = jnp.pad(cols, (0, pad))
    laddrp = jnp.pad(laddr, (0, pad))
    wp = jnp.pad(weights, (0, pad))

    bounds = jnp.arange(0, N_NEURONS + 1, RB, dtype=jnp.int32)
    offs = jnp.zeros((72,), jnp.int32).at[:NBLK + 1].set(
        jnp.searchsorted(rows, bounds).astype(jnp.int32))

    mesh = plsc.VectorSubcoreMesh(core_axis_name="c", subcore_axis_name="s")
    y3 = pl.kernel(
        _sc_body,
        mesh=mesh,
        out_type=jax.ShapeDtypeStruct((N_SYN, T, N_NEURONS), jnp.float32),
        scratch_types=[
            pltpu.SMEM((72,), jnp.int32),
            pltpu.VMEM((SG, 128), jnp.int32),
            pltpu.VMEM((S,), jnp.int32),
            pltpu.VMEM((S,), jnp.float32),
            pltpu.VMEM((SG, 128, C), jnp.float32),
            pltpu.VMEM((C, N_SYN * RB), jnp.float32),
            pltpu.SemaphoreType.DMA,
        ],
    )(xcc, colsp, laddrp, wp, offs)

    nb_tc = 2048
    out5 = pl.pallas_call(
        _tc_contract_kernel,
        grid=(N_NEURONS // nb_tc,),
        in_specs=[
            pl.BlockSpec((N_SYN, T, nb_tc), lambda i: (0, 0, i)),
            pl.BlockSpec(memory_space=pltpu.SMEM),
        ],
        out_specs=pl.BlockSpec((N_BASIS, T, nb_tc), lambda i: (0, 0, i)),
        out_shape=jax.ShapeDtypeStruct((N_BASIS, T, N_NEURONS), jnp.float32),
    )(y3, synaptic_weights)

    return jnp.transpose(out5, (1, 2, 0)).reshape(b, t, N_NEURONS * N_BASIS)


# speedup vs baseline: 3.1617x; 3.1617x over previous
import functools

import jax
import jax.numpy as jnp
from jax import lax
from jax.experimental import pallas as pl
from jax.experimental.pallas import tpu as pltpu
from jax.experimental.pallas import tpu_sc as plsc

N_NEURONS = 16384
N_IN = 16384
NNZ = 268435
N_BASIS = 5
N_SYN = 10
T = 256

RB = 256
NBLK = N_NEURONS // RB
NWORK = 32
PASSES = NBLK // NWORK
C = 32
NBC = T // C
S = 1024
SG = 8
NNZ_P = ((NNZ + 8 + S - 1) // S + 1) * S
ACCW = N_SYN * C * RB


def _lane_bcast(v, i):
    return lax.gather(
        v,
        jnp.full((16, 1), i, dtype=jnp.int32),
        lax.GatherDimensionNumbers(
            offset_dims=(), collapsed_slice_dims=(0,), start_index_map=(0,)),
        slice_sizes=(1,),
        mode=lax.GatherScatterMode.PROMISE_IN_BOUNDS)


def _sc_body(xcc, colsp, laddrp, wp, offs, y3,
             offs_s, offs_sp, cidx, la_v, w_v, xbuf, acc, sem):
    sid = lax.axis_index("s")
    wid = lax.axis_index("c") * 16 + sid

    @pl.when(sid == 0)
    def _():
        pltpu.sync_copy(offs, offs_sp)
    plsc.subcore_barrier()
    pltpu.sync_copy(offs_sp, offs_s)
    ciota = lax.iota(jnp.int32, 16)
    ciota_sc = [ciota * RB, (ciota + 16) * RB]

    def pass_body(p, _):
        blk = wid * PASSES + p
        p0 = offs_s[blk]
        p1 = offs_s[blk + 1]
        p0a = p0 & ~7
        nch = (p1 - p0a + S - 1) // S

        def nb_body(nb, _):
            def zr(z, _):
                acc[pl.ds(z * 16, 16)] = jnp.zeros((16,), jnp.float32)
                return 0
            lax.fori_loop(0, ACCW // 16, zr, 0)

            nbase = nb * N_IN

            def ch_body(ch, _):
                g0 = pl.multiple_of(p0a + ch * S, 8)
                for k in range(SG):
                    pltpu.sync_copy(colsp.at[pl.ds(g0 + k * 128, 128)],
                                    cidx.at[k])
                pltpu.sync_copy(laddrp.at[pl.ds(g0, S)], la_v)
                pltpu.sync_copy(wp.at[pl.ds(g0, S)], w_v)

                def fix(j, _):
                    sl = pl.ds(j * 16, 16)
                    gidx = g0 + j * 16 + ciota
                    ok = (gidx >= p0) & (gidx < p1)
                    w_v[sl] = jnp.where(ok, w_v[sl], 0.0)
                    k = j // 8
                    j2 = j % 8
                    sl2 = pl.ds(j2 * 16, 16)
                    cidx[k, sl2] = cidx[k, sl2] + nbase
                    return 0
                lax.fori_loop(0, S // 16, fix, 0)

                cps = [pltpu.async_copy(xcc.at[cidx.at[k]], xbuf.at[k], sem)
                       for k in range(SG)]
                for cp in cps:
                    cp.wait()

                def grp(k, _):
                    def grp2(jj, _):
                        base = k * 128 + jj * 16
                        a16 = la_v[pl.ds(base, 16)]
                        w16 = w_v[pl.ds(base, 16)]
                        for i in range(16):
                            ab = _lane_bcast(a16, i)
                            wb = _lane_bcast(w16, i)
                            for cc in range(2):
                                xrow = xbuf[k, jj * 16 + i, pl.ds(cc * 16, 16)]
                                plsc.addupdate_scatter(
                                    acc, [ab + ciota_sc[cc]], wb * xrow)
                        return 0
                    return lax.fori_loop(0, 8, grp2, 0)
                lax.fori_loop(0, SG, grp, 0)
                return 0
            lax.fori_loop(0, nch, ch_body, 0)

            for s in range(N_SYN):
                dst0 = ((s * NBC + nb) * NBLK + blk) * (C * RB)
                pltpu.sync_copy(
                    acc.at[pl.ds(s * C * RB, C * RB)],
                    y3.at[pl.ds(dst0, C * RB)])
            return 0
        lax.fori_loop(0, NBC, nb_body, 0)
        return 0
    lax.fori_loop(0, PASSES, pass_body, 0)


def _tc_contract_kernel(y_ref, f_ref, o_ref):
    for r in range(N_BASIS):
        z = f_ref[0, r] * y_ref[0, :, 0]
        for s in range(1, N_SYN):
            z = z + f_ref[s, r] * y_ref[s, :, 0]
        o_ref[r, 0] = z


def kernel(inp, weights, synaptic_weights, indices, syn_ids):
    b, t, n_in = inp.shape
    rows = indices[:, 0].astype(jnp.int32)
    cols = indices[:, 1].astype(jnp.int32)

    xT = inp.reshape(T, N_IN)
    xcc = xT.reshape(NBC, C, N_IN).transpose(0, 2, 1).reshape(NBC * N_IN, C)

    laddr = syn_ids.astype(jnp.int32) * (C * RB) + (rows % RB)
    pad = NNZ_P - NNZ
    colsp = jnp.pad(cols, (0, pad))
    laddrp = jnp.pad(laddr, (0, pad))
    wp = jnp.pad(weights, (0, pad))

    bounds = jnp.arange(0, N_NEURONS + 1, RB, dtype=jnp.int32)
    offs = jnp.zeros((72,), jnp.int32).at[:NBLK + 1].set(
        jnp.searchsorted(rows, bounds).astype(jnp.int32))

    mesh = plsc.VectorSubcoreMesh(core_axis_name="c", subcore_axis_name="s")
    y3 = pl.kernel(
        _sc_body,
        mesh=mesh,
        compiler_params=pltpu.CompilerParams(needs_layout_passes=False, use_tc_tiling_on_sc=False),
        out_type=jax.ShapeDtypeStruct((N_SYN * NBC * NBLK * C * RB,), jnp.float32),
        scratch_types=[
            pltpu.SMEM((72,), jnp.int32),
            pltpu.VMEM_SHARED((72,), jnp.int32),
            pltpu.VMEM((SG, 128), jnp.int32),
            pltpu.VMEM((S,), jnp.int32),
            pltpu.VMEM((S,), jnp.float32),
            pltpu.VMEM((SG, 128, C), jnp.float32),
            pltpu.VMEM((ACCW,), jnp.float32),
            pltpu.SemaphoreType.DMA,
        ],
    )(xcc, colsp, laddrp, wp, offs)

    y5 = y3.reshape(N_SYN, NBC, NBLK, C, RB)
    out6 = pl.pallas_call(
        _tc_contract_kernel,
        grid=(NBLK,),
        in_specs=[
            pl.BlockSpec((N_SYN, NBC, 1, C, RB), lambda i: (0, 0, i, 0, 0)),
            pl.BlockSpec(memory_space=pltpu.SMEM),
        ],
        out_specs=pl.BlockSpec((N_BASIS, 1, NBC, C, RB),
                               lambda i: (0, i, 0, 0, 0)),
        out_shape=jax.ShapeDtypeStruct((N_BASIS, NBLK, NBC, C, RB),
                                       jnp.float32),
    )(y5, synaptic_weights)

    return jnp.transpose(out6, (2, 3, 1, 4, 0)).reshape(
        b, t, N_NEURONS * N_BASIS)
